# baseline (speedup 1.0000x reference)
import functools

import jax
import jax.numpy as jnp
from jax import lax
from jax.experimental import pallas as pl
from jax.experimental.pallas import tpu as pltpu
from jax.experimental.pallas import tpu_sc as plsc


def kernel(x, branch_idx, weight, bias):
    T, D = x.shape
    NB = weight.shape[0]
    idx = branch_idx.astype(jnp.int32)

    info = plsc.get_sparse_core_info()
    NC, NS, L = info.num_cores, info.num_subcores, info.num_lanes
    tpc = T // NC
    CS = D // NS
    wt = weight.reshape(NB, NS, CS).transpose(1, 0, 2).reshape(-1)
    btr = bias.reshape(NB, NS, CS).transpose(1, 0, 2)
    NT = 128
    nblk = tpc // NT

    mesh = plsc.VectorSubcoreMesh(core_axis_name="c", subcore_axis_name="s")

    @functools.partial(
        pl.kernel,
        mesh=mesh,
        compiler_params=pltpu.CompilerParams(needs_layout_passes=False),
        out_type=jax.ShapeDtypeStruct((T, D), jnp.float32),
        scratch_types=[
            pltpu.VMEM((tpc,), jnp.int32),
            pltpu.VMEM((CS * NB,), jnp.float32),
            [pltpu.VMEM((NT, CS), jnp.float32) for _ in range(2)],
            [pltpu.VMEM((NT, CS), jnp.float32) for _ in range(2)],
            pltpu.SemaphoreType.DMA((2,)),
            pltpu.SemaphoreType.DMA((2,)),
            pltpu.SemaphoreType.DMA((2,)),
        ],
    )
    def run(x_hbm, idx_hbm, wt_hbm, b_hbm, out_hbm, idx_v, w_v, xb, ob,
            x_sem, o_sem, a_sem):
        cid = lax.axis_index("c")
        sid = lax.axis_index("s")
        tbase = pl.multiple_of(cid * tpc, 8)
        cs = pl.multiple_of(sid * CS, L)
        tab0 = pl.multiple_of(sid * (CS * NB), 8)

        pltpu.sync_copy(wt_hbm.at[pl.ds(tab0, CS * NB)], w_v)
        pltpu.sync_copy(idx_hbm.at[pl.ds(tbase, tpc)], idx_v)

        def issue_in(j, p):
            pltpu.async_copy(
                x_hbm.at[pl.ds(tbase + j * NT, NT), pl.ds(cs, CS)], xb[p],
                x_sem.at[p])

        def issue_badd(j, p):
            pltpu.async_copy(
                b_hbm.at[sid].at[idx_v.at[pl.ds(j * NT, NT)]], ob[p],
                a_sem.at[p], add=True)

        def issue_out(j, p):
            pltpu.async_copy(
                ob[p], out_hbm.at[pl.ds(tbase + j * NT, NT), pl.ds(cs, CS)],
                o_sem.at[p])

        issue_in(0, 0)

        def outer(j2, carry):
            for p in range(2):
                j = j2 * 2 + p
                pltpu.make_async_copy(
                    x_hbm.at[pl.ds(tbase + j * NT, NT), pl.ds(cs, CS)],
                    xb[p], x_sem.at[p]).wait()

                @pl.when(j + 1 < nblk)
                def _():
                    issue_in(j + 1, 1 - p)

                @pl.when(j >= 2)
                def _():
                    pltpu.make_async_copy(
                        ob[p],
                        out_hbm.at[
                            pl.ds(tbase + (j - 2) * NT, NT), pl.ds(cs, CS)],
                        o_sem.at[p]).wait()

                tb = j * NT

                @plsc.parallel_loop(0, NT // L, unroll=2)
                def _(g):
                    iv = idx_v[pl.ds(tb + g * L, L)]
                    for tt in range(L):
                        bio = iv[tt] * CS
                        t = g * L + tt
                        for k in range(CS // L):
                            sl = pl.ds(k * L, L)
                            wv = w_v[pl.ds(bio + k * L, L)]
                            ob[p][t, sl] = xb[p][t, sl] * wv

                issue_badd(j, p)

                @pl.when(j >= 1)
                def _():
                    pltpu.make_async_copy(
                        b_hbm.at[sid].at[idx_v.at[pl.ds((j - 1) * NT, NT)]],
                        ob[1 - p], a_sem.at[1 - p]).wait()
                    issue_out(j - 1, 1 - p)
            return carry

        lax.fori_loop(0, nblk // 2, outer, 0)

        lastp = (nblk - 1) % 2
        pltpu.make_async_copy(
            b_hbm.at[sid].at[idx_v.at[pl.ds((nblk - 1) * NT, NT)]],
            ob[lastp], a_sem.at[lastp]).wait()
        issue_out(nblk - 1, lastp)
        for j in (nblk - 2, nblk - 1):
            p = j % 2
            pltpu.make_async_copy(
                ob[p], out_hbm.at[pl.ds(tbase + j * NT, NT), pl.ds(cs, CS)],
                o_sem.at[p]).wait()

    return run(x, idx, wt, btr)


# speedup vs baseline: 1.3609x; 1.3609x over previous
import functools

import jax
import jax.numpy as jnp
from jax import lax
from jax.experimental import pallas as pl
from jax.experimental.pallas import tpu as pltpu
from jax.experimental.pallas import tpu_sc as plsc


def kernel(x, branch_idx, weight, bias):
    T, D = x.shape
    NB = weight.shape[0]
    idx = branch_idx.astype(jnp.int32)

    info = plsc.get_sparse_core_info()
    NC, NS, L = info.num_cores, info.num_subcores, info.num_lanes
    tpc = T // NC
    CS = D // NS
    wt = weight.reshape(NB, NS, CS).transpose(1, 0, 2).reshape(-1)
    btr = bias.reshape(NB, NS, CS).transpose(1, 0, 2)
    NT = 128
    nblk = tpc // NT

    mesh = plsc.VectorSubcoreMesh(core_axis_name="c", subcore_axis_name="s")

    @functools.partial(
        pl.kernel,
        mesh=mesh,
        compiler_params=pltpu.CompilerParams(needs_layout_passes=False),
        out_type=jax.ShapeDtypeStruct((T, D), jnp.float32),
        scratch_types=[
            pltpu.VMEM((tpc,), jnp.int32),
            pltpu.VMEM((CS * NB,), jnp.float32),
            [pltpu.VMEM((NT, CS), jnp.float32) for _ in range(2)],
            [pltpu.VMEM((NT, CS), jnp.float32) for _ in range(2)],
            pltpu.SemaphoreType.DMA((2,)),
            pltpu.SemaphoreType.DMA((2,)),
            pltpu.SemaphoreType.DMA((2,)),
        ],
    )
    def run(x_hbm, idx_hbm, wt_hbm, b_hbm, out_hbm, idx_v, w_v, xb, ob,
            x_sem, o_sem, a_sem):
        cid = lax.axis_index("c")
        sid = lax.axis_index("s")
        tbase = pl.multiple_of(cid * tpc, 8)
        cs = pl.multiple_of(sid * CS, L)
        tab0 = pl.multiple_of(sid * (CS * NB), 8)

        pltpu.sync_copy(wt_hbm.at[pl.ds(tab0, CS * NB)], w_v)
        pltpu.sync_copy(idx_hbm.at[pl.ds(tbase, tpc)], idx_v)

        def issue_in(j, p):
            pltpu.async_copy(
                x_hbm.at[pl.ds(tbase + j * NT, NT), pl.ds(cs, CS)], xb[p],
                x_sem.at[p])

        def issue_badd(j, p):
            pltpu.async_copy(
                b_hbm.at[sid].at[idx_v.at[pl.ds(j * NT, NT)]], ob[p],
                a_sem.at[p], add=True)

        def issue_out(j, p):
            pltpu.async_copy(
                ob[p], out_hbm.at[pl.ds(tbase + j * NT, NT), pl.ds(cs, CS)],
                o_sem.at[p])

        issue_in(0, 0)

        def outer(j2, carry):
            for p in range(2):
                j = j2 * 2 + p
                pltpu.make_async_copy(
                    x_hbm.at[pl.ds(tbase + j * NT, NT), pl.ds(cs, CS)],
                    xb[p], x_sem.at[p]).wait()

                @pl.when(j + 1 < nblk)
                def _():
                    issue_in(j + 1, 1 - p)

                @pl.when(j >= 2)
                def _():
                    pltpu.make_async_copy(
                        ob[p],
                        out_hbm.at[
                            pl.ds(tbase + (j - 2) * NT, NT), pl.ds(cs, CS)],
                        o_sem.at[p]).wait()

                tb = j * NT

                @plsc.parallel_loop(0, NT // L, unroll=1)
                def _(g):
                    ivs = idx_v[pl.ds(tb + g * L, L)] * CS
                    for tt in range(L):
                        bio = ivs[tt]
                        t = g * L + tt
                        for k in range(CS // L):
                            sl = pl.ds(k * L, L)
                            wv = w_v[pl.ds(bio + k * L, L)]
                            ob[p][t, sl] = xb[p][t, sl] * wv

                issue_badd(j, p)

                @pl.when(j >= 1)
                def _():
                    pltpu.make_async_copy(
                        b_hbm.at[sid].at[idx_v.at[pl.ds((j - 1) * NT, NT)]],
                        ob[1 - p], a_sem.at[1 - p]).wait()
                    issue_out(j - 1, 1 - p)
            return carry

        lax.fori_loop(0, nblk // 2, outer, 0)

        lastp = (nblk - 1) % 2
        pltpu.make_async_copy(
            b_hbm.at[sid].at[idx_v.at[pl.ds((nblk - 1) * NT, NT)]],
            ob[lastp], a_sem.at[lastp]).wait()
        issue_out(nblk - 1, lastp)
        for j in (nblk - 2, nblk - 1):
            p = j % 2
            pltpu.make_async_copy(
                ob[p], out_hbm.at[pl.ds(tbase + j * NT, NT), pl.ds(cs, CS)],
                o_sem.at[p]).wait()

    return run(x, idx, wt, btr)
